# Initial kernel scaffold; baseline (speedup 1.0000x reference)
import jax
import jax.numpy as jnp
from jax.experimental import pallas as pl


def kernel(A_p, A_n, x_p, x_n, w_p, w_n):
    raise NotImplementedError("write your pallas kernel here")


# speedup vs baseline: 1.6297x; 1.6297x over previous
import functools

import jax
import jax.numpy as jnp
from jax.experimental import pallas as pl


_BM = 512


def _mm_kernel(a_ref, x_ref, o_ref):
    o_ref[...] = jax.lax.dot_general(
        a_ref[...], x_ref[...],
        (((1,), (0,)), ((), ())),
        preferred_element_type=jnp.float32,
    )


def _mm_bias_kernel(a_ref, x_ref, b_ref, o_ref):
    o_ref[...] = b_ref[...] + jax.lax.dot_general(
        a_ref[...], x_ref[...],
        (((1,), (0,)), ((), ())),
        preferred_element_type=jnp.float32,
    )


@functools.partial(jax.jit, static_argnames=())
def _pass_mm(A, X):
    N, K = A.shape
    F = X.shape[1]
    return pl.pallas_call(
        _mm_kernel,
        grid=(N // _BM,),
        in_specs=[
            pl.BlockSpec((_BM, K), lambda i: (i, 0)),
            pl.BlockSpec((K, F), lambda i: (0, 0)),
        ],
        out_specs=pl.BlockSpec((_BM, F), lambda i: (i, 0)),
        out_shape=jax.ShapeDtypeStruct((N, F), jnp.float32),
    )(A, X)


@functools.partial(jax.jit, static_argnames=())
def _pass_mm_bias(A, X, B):
    N, K = A.shape
    F = X.shape[1]
    return pl.pallas_call(
        _mm_bias_kernel,
        grid=(N // _BM,),
        in_specs=[
            pl.BlockSpec((_BM, K), lambda i: (i, 0)),
            pl.BlockSpec((K, F), lambda i: (0, 0)),
            pl.BlockSpec((_BM, F), lambda i: (i, 0)),
        ],
        out_specs=pl.BlockSpec((_BM, F), lambda i: (i, 0)),
        out_shape=jax.ShapeDtypeStruct((N, F), jnp.float32),
    )(A, X, B)


def kernel(A_p, A_n, x_p, x_n, w_p, w_n):
    D = x_p.shape[1]

    X1 = jnp.concatenate([x_p, x_n], axis=1)
    Y1 = _pass_mm(A_p, X1)
    y1, t1 = Y1[:, :D], Y1[:, D:]

    X2 = jnp.concatenate([x_n, t1], axis=1)
    Y2 = _pass_mm(A_n, X2)
    z1, t2 = Y2[:, :D], Y2[:, D:]

    X3 = jnp.concatenate([w_p[2] * y1, w_n[1] * z1], axis=1)
    PQ = jnp.concatenate(
        [w_p[0] * x_p + w_p[1] * y1, w_n[0] * z1 + w_n[2] * t2], axis=1)
    return _pass_mm_bias(A_p, X3, PQ)


# speedup vs baseline: 1.7377x; 1.0663x over previous
import functools

import jax
import jax.numpy as jnp
from jax.experimental import pallas as pl


_BM = 512


def _mm_kernel(a_ref, x_ref, o_ref):
    o_ref[...] = jax.lax.dot_general(
        a_ref[...].astype(jnp.bfloat16), x_ref[...],
        (((1,), (0,)), ((), ())),
        preferred_element_type=jnp.float32,
    )


def _mm_bias_kernel(a_ref, x_ref, b_ref, o_ref):
    o_ref[...] = b_ref[...] + jax.lax.dot_general(
        a_ref[...].astype(jnp.bfloat16), x_ref[...],
        (((1,), (0,)), ((), ())),
        preferred_element_type=jnp.float32,
    )


@functools.partial(jax.jit, static_argnames=())
def _pass_mm(A, X):
    N, K = A.shape
    F = X.shape[1]
    return pl.pallas_call(
        _mm_kernel,
        grid=(N // _BM,),
        in_specs=[
            pl.BlockSpec((_BM, K), lambda i: (i, 0)),
            pl.BlockSpec((K, F), lambda i: (0, 0)),
        ],
        out_specs=pl.BlockSpec((_BM, F), lambda i: (i, 0)),
        out_shape=jax.ShapeDtypeStruct((N, F), jnp.float32),
    )(A, X)


@functools.partial(jax.jit, static_argnames=())
def _pass_mm_bias(A, X, B):
    N, K = A.shape
    F = X.shape[1]
    return pl.pallas_call(
        _mm_bias_kernel,
        grid=(N // _BM,),
        in_specs=[
            pl.BlockSpec((_BM, K), lambda i: (i, 0)),
            pl.BlockSpec((K, F), lambda i: (0, 0)),
            pl.BlockSpec((_BM, F), lambda i: (i, 0)),
        ],
        out_specs=pl.BlockSpec((_BM, F), lambda i: (i, 0)),
        out_shape=jax.ShapeDtypeStruct((N, F), jnp.float32),
    )(A, X, B)


def kernel(A_p, A_n, x_p, x_n, w_p, w_n):
    D = x_p.shape[1]

    X1 = jnp.concatenate([x_p, x_n], axis=1).astype(jnp.bfloat16)
    Y1 = _pass_mm(A_p, X1)
    y1, t1 = Y1[:, :D], Y1[:, D:]

    X2 = jnp.concatenate([x_n, t1], axis=1).astype(jnp.bfloat16)
    Y2 = _pass_mm(A_n, X2)
    z1, t2 = Y2[:, :D], Y2[:, D:]

    X3 = jnp.concatenate(
        [w_p[2] * y1, w_n[1] * z1], axis=1).astype(jnp.bfloat16)
    PQ = jnp.concatenate(
        [w_p[0] * x_p + w_p[1] * y1, w_n[0] * z1 + w_n[2] * t2], axis=1)
    return _pass_mm_bias(A_p, X3, PQ)


# speedup vs baseline: 1.7807x; 1.0247x over previous
import functools

import jax
import jax.numpy as jnp
from jax.experimental import pallas as pl


_BM = 256


def _mm_kernel(a_ref, x_ref, o_ref):
    o_ref[...] = jax.lax.dot_general(
        a_ref[...].astype(jnp.bfloat16), x_ref[...],
        (((1,), (0,)), ((), ())),
        preferred_element_type=jnp.float32,
    )


def _mm_bias_kernel(a_ref, x_ref, b_ref, o_ref):
    o_ref[...] = b_ref[...] + jax.lax.dot_general(
        a_ref[...].astype(jnp.bfloat16), x_ref[...],
        (((1,), (0,)), ((), ())),
        preferred_element_type=jnp.float32,
    )


@functools.partial(jax.jit, static_argnames=())
def _pass_mm(A, X):
    N, K = A.shape
    F = X.shape[1]
    return pl.pallas_call(
        _mm_kernel,
        grid=(N // _BM,),
        in_specs=[
            pl.BlockSpec((_BM, K), lambda i: (i, 0)),
            pl.BlockSpec((K, F), lambda i: (0, 0)),
        ],
        out_specs=pl.BlockSpec((_BM, F), lambda i: (i, 0)),
        out_shape=jax.ShapeDtypeStruct((N, F), jnp.float32),
    )(A, X)


@functools.partial(jax.jit, static_argnames=())
def _pass_mm_bias(A, X, B):
    N, K = A.shape
    F = X.shape[1]
    return pl.pallas_call(
        _mm_bias_kernel,
        grid=(N // _BM,),
        in_specs=[
            pl.BlockSpec((_BM, K), lambda i: (i, 0)),
            pl.BlockSpec((K, F), lambda i: (0, 0)),
            pl.BlockSpec((_BM, F), lambda i: (i, 0)),
        ],
        out_specs=pl.BlockSpec((_BM, F), lambda i: (i, 0)),
        out_shape=jax.ShapeDtypeStruct((N, F), jnp.float32),
    )(A, X, B)


def kernel(A_p, A_n, x_p, x_n, w_p, w_n):
    D = x_p.shape[1]

    X1 = jnp.concatenate([x_p, x_n], axis=1).astype(jnp.bfloat16)
    Y1 = _pass_mm(A_p, X1)
    y1, t1 = Y1[:, :D], Y1[:, D:]

    X2 = jnp.concatenate([x_n, t1], axis=1).astype(jnp.bfloat16)
    Y2 = _pass_mm(A_n, X2)
    z1, t2 = Y2[:, :D], Y2[:, D:]

    X3 = jnp.concatenate(
        [w_p[2] * y1, w_n[1] * z1], axis=1).astype(jnp.bfloat16)
    PQ = jnp.concatenate(
        [w_p[0] * x_p + w_p[1] * y1, w_n[0] * z1 + w_n[2] * t2], axis=1)
    return _pass_mm_bias(A_p, X3, PQ)
